# baseline (speedup 1.0000x reference)
import functools

import jax
import jax.numpy as jnp
import numpy as np
from jax import lax
from jax.experimental import pallas as pl
from jax.experimental.pallas import tpu as pltpu
from jax.experimental.pallas import tpu_sc as plsc

DIM = 17
D2 = DIM * DIM
D3 = DIM * DIM * DIM
NLUT = 2 * D3
B = 16
W = 512
PIX = W * W
CHUNK = 4096
ROWS_PER_CHUNK = CHUNK // W
NWORKERS = 32
CHUNKS_PER_BATCH = PIX // CHUNK
TOTAL_CHUNKS = B * CHUNKS_PER_BATCH
CHUNKS_PER_WORKER = TOTAL_CHUNKS // NWORKERS

_RSCALE = np.float32((DIM - 1) / 1.000001)
_URSCALE = np.float32(1.0 / 1.000001)


def _compute_chunk(lutR, lutG, lutB, xin, yout):
    def row_body(rr, carry2):
        @plsc.parallel_loop(0, W // 16, unroll=2)
        def px(j):
            s = pl.ds(j * 16, 16)
            u = xin[0, rr, s]
            r = xin[1, rr, s]
            g = xin[2, rr, s]
            bl = xin[3, rr, s]
            rf = r * _RSCALE
            gf = g * _RSCALE
            bf = bl * _RSCALE
            ri = jnp.clip(rf.astype(jnp.int32), 0, DIM - 2)
            gi = jnp.clip(gf.astype(jnp.int32), 0, DIM - 2)
            bi = jnp.clip(bf.astype(jnp.int32), 0, DIM - 2)
            dr = rf - ri.astype(jnp.float32)
            dg = gf - gi.astype(jnp.float32)
            db = bf - bi.astype(jnp.float32)
            du = u * _URSCALE
            base = ri * D2 + gi * DIM + bi
            w_k = (1.0 - db, db)
            w_j = (1.0 - dg, dg)
            w_i = (1.0 - dr, dr)
            acc = [[None] * 3, [None] * 3]
            for di in (0, 1):
                for dj in (0, 1):
                    wij = w_i[di] * w_j[dj]
                    for dk in (0, 1):
                        wijk = wij * w_k[dk]
                        idx0 = base + (di * D2 + dj * DIM + dk)
                        idx1 = idx0 + D3
                        for p, idx in ((0, idx0), (1, idx1)):
                            for c, lut in enumerate((lutR, lutG, lutB)):
                                v = wijk * plsc.load_gather(lut, [idx])
                                a = acc[p][c]
                                acc[p][c] = v if a is None else a + v
            omu = 1.0 - du
            yout[0, rr, s] = omu * acc[0][0] + du * acc[1][0]
            yout[1, rr, s] = omu * acc[0][1] + du * acc[1][1]
            yout[2, rr, s] = omu * acc[0][2] + du * acc[1][2]
        return carry2

    lax.fori_loop(0, ROWS_PER_CHUNK, row_body, 0)


def _lut_body(x_h, lr_h, lg_h, lb_h, out_h,
              lutR, lutG, lutB, xin0, xin1, yout0, yout1,
              si0, si1, so0, so1):
    pltpu.sync_copy(lr_h, lutR)
    pltpu.sync_copy(lg_h, lutG)
    pltpu.sync_copy(lb_h, lutB)

    cid = lax.axis_index("c")
    sid = lax.axis_index("s")
    wid = sid * 2 + cid
    t_base = wid * CHUNKS_PER_WORKER

    def src_in(t):
        b = t // CHUNKS_PER_BATCH
        row0 = (t % CHUNKS_PER_BATCH) * ROWS_PER_CHUNK
        return x_h.at[b, :, pl.ds(row0, ROWS_PER_CHUNK), :]

    def dst_out(t):
        b = t // CHUNKS_PER_BATCH
        row0 = (t % CHUNKS_PER_BATCH) * ROWS_PER_CHUNK
        return out_h.at[b, :, pl.ds(row0, ROWS_PER_CHUNK), :]

    pltpu.async_copy(src_in(t_base), xin0, si0)

    def pair_body(m, carry):
        t0 = t_base + 2 * m
        pltpu.async_copy(src_in(t0 + 1), xin1, si1)
        pltpu.make_async_copy(src_in(t0), xin0, si0).wait()

        @pl.when(m > 0)
        def _():
            pltpu.make_async_copy(yout0, dst_out(t0), so0).wait()

        _compute_chunk(lutR, lutG, lutB, xin0, yout0)
        pltpu.async_copy(yout0, dst_out(t0), so0)

        t2 = jnp.minimum(t0 + 2, t_base + CHUNKS_PER_WORKER - 1)
        pltpu.async_copy(src_in(t2), xin0, si0)
        pltpu.make_async_copy(src_in(t0 + 1), xin1, si1).wait()

        @pl.when(m > 0)
        def _():
            pltpu.make_async_copy(yout1, dst_out(t0), so1).wait()

        _compute_chunk(lutR, lutG, lutB, xin1, yout1)
        pltpu.async_copy(yout1, dst_out(t0 + 1), so1)
        return carry

    lax.fori_loop(0, CHUNKS_PER_WORKER // 2, pair_body, 0)
    pltpu.make_async_copy(src_in(t_base), xin0, si0).wait()
    pltpu.make_async_copy(yout0, dst_out(t_base), so0).wait()
    pltpu.make_async_copy(yout1, dst_out(t_base), so1).wait()


_mesh = plsc.VectorSubcoreMesh(core_axis_name="c", subcore_axis_name="s")

_lut_apply = functools.partial(
    pl.kernel,
    out_type=jax.ShapeDtypeStruct((B, 3, W, W), jnp.float32),
    mesh=_mesh,
    scratch_types=[
        pltpu.VMEM((NLUT,), jnp.float32),
        pltpu.VMEM((NLUT,), jnp.float32),
        pltpu.VMEM((NLUT,), jnp.float32),
        pltpu.VMEM((4, ROWS_PER_CHUNK, W), jnp.float32),
        pltpu.VMEM((4, ROWS_PER_CHUNK, W), jnp.float32),
        pltpu.VMEM((3, ROWS_PER_CHUNK, W), jnp.float32),
        pltpu.VMEM((3, ROWS_PER_CHUNK, W), jnp.float32),
        pltpu.SemaphoreType.DMA,
        pltpu.SemaphoreType.DMA,
        pltpu.SemaphoreType.DMA,
        pltpu.SemaphoreType.DMA,
    ],
    compiler_params=pltpu.CompilerParams(needs_layout_passes=False),
)(_lut_body)


@jax.jit
def kernel(x, LUT_en):
    lutf = LUT_en.reshape(3, NLUT)
    return _lut_apply(x, lutf[0], lutf[1], lutf[2])


# speedup vs baseline: 1.0695x; 1.0695x over previous
import functools

import jax
import jax.numpy as jnp
import numpy as np
from jax import lax
from jax.experimental import pallas as pl
from jax.experimental.pallas import tpu as pltpu
from jax.experimental.pallas import tpu_sc as plsc

DIM = 17
D2 = DIM * DIM
D3 = DIM * DIM * DIM
NLUT = 2 * D3
B = 16
W = 512
PIX = W * W
CHUNK = 4096
ROWS_PER_CHUNK = CHUNK // W
NWORKERS = 32
CHUNKS_PER_BATCH = PIX // CHUNK
TOTAL_CHUNKS = B * CHUNKS_PER_BATCH
CHUNKS_PER_WORKER = TOTAL_CHUNKS // NWORKERS

_RSCALE = np.float32((DIM - 1) / 1.000001)
_URSCALE = np.float32(1.0 / 1.000001)


def _compute_chunk(lutR, lutG, lutB, xin, yout):
    def row_body(rr, carry2):
        @plsc.parallel_loop(0, W // 16, unroll=2)
        def px(j):
            s = pl.ds(j * 16, 16)
            u = xin[0, rr, s]
            r = xin[1, rr, s]
            g = xin[2, rr, s]
            bl = xin[3, rr, s]
            rf = r * _RSCALE
            gf = g * _RSCALE
            bf = bl * _RSCALE
            ri = jnp.clip(rf.astype(jnp.int32), 0, DIM - 2)
            gi = jnp.clip(gf.astype(jnp.int32), 0, DIM - 2)
            bi = jnp.clip(bf.astype(jnp.int32), 0, DIM - 2)
            dr = rf - ri.astype(jnp.float32)
            dg = gf - gi.astype(jnp.float32)
            db = bf - bi.astype(jnp.float32)
            du = u * _URSCALE
            base = ri * D2 + gi * DIM + bi
            w_k = (1.0 - db, db)
            w_j = (1.0 - dg, dg)
            w_i = (1.0 - dr, dr)
            acc = [[None] * 3, [None] * 3]
            for di in (0, 1):
                for dj in (0, 1):
                    wij = w_i[di] * w_j[dj]
                    for dk in (0, 1):
                        wijk = wij * w_k[dk]
                        idx0 = base + (di * D2 + dj * DIM + dk)
                        idx1 = idx0 + D3
                        for p, idx in ((0, idx0), (1, idx1)):
                            for c, lut in enumerate((lutR, lutG, lutB)):
                                v = wijk * plsc.load_gather(lut, [idx])
                                a = acc[p][c]
                                acc[p][c] = v if a is None else a + v
            omu = 1.0 - du
            yout[0, rr, s] = omu * acc[0][0] + du * acc[1][0]
            yout[1, rr, s] = omu * acc[0][1] + du * acc[1][1]
            yout[2, rr, s] = omu * acc[0][2] + du * acc[1][2]
        return carry2

    lax.fori_loop(0, ROWS_PER_CHUNK, row_body, 0)


def _lut_body(x_h, lr_h, lg_h, lb_h, out_h,
              lutR, lutG, lutB, xin, yout, si, so):
    pltpu.sync_copy(lr_h, lutR)
    pltpu.sync_copy(lg_h, lutG)
    pltpu.sync_copy(lb_h, lutB)

    cid = lax.axis_index("c")
    sid = lax.axis_index("s")
    wid = sid * 2 + cid
    t_base = wid * CHUNKS_PER_WORKER

    def src_in(t):
        b = t // CHUNKS_PER_BATCH
        row0 = (t % CHUNKS_PER_BATCH) * ROWS_PER_CHUNK
        return x_h.at[b, :, pl.ds(row0, ROWS_PER_CHUNK), :]

    def dst_out(t):
        b = t // CHUNKS_PER_BATCH
        row0 = (t % CHUNKS_PER_BATCH) * ROWS_PER_CHUNK
        return out_h.at[b, :, pl.ds(row0, ROWS_PER_CHUNK), :]

    pltpu.async_copy(src_in(t_base), xin.at[0], si.at[0])

    def body(tl, carry):
        t = t_base + tl
        par = tl & 1
        npar = 1 - par
        tnext = t_base + jnp.minimum(tl + 1, CHUNKS_PER_WORKER - 1)
        pltpu.async_copy(src_in(tnext), xin.at[npar], si.at[npar])
        pltpu.make_async_copy(src_in(t), xin.at[par], si.at[par]).wait()

        @pl.when(tl > 1)
        def _():
            pltpu.make_async_copy(yout.at[par], dst_out(t), so.at[par]).wait()

        _compute_chunk(lutR, lutG, lutB, xin.at[par], yout.at[par])
        pltpu.async_copy(yout.at[par], dst_out(t), so.at[par])
        return carry

    lax.fori_loop(0, CHUNKS_PER_WORKER, body, 0)
    pltpu.make_async_copy(src_in(t_base), xin.at[0], si.at[0]).wait()
    pltpu.make_async_copy(yout.at[0], dst_out(t_base), so.at[0]).wait()
    pltpu.make_async_copy(yout.at[1], dst_out(t_base), so.at[1]).wait()


_mesh = plsc.VectorSubcoreMesh(core_axis_name="c", subcore_axis_name="s")

_lut_apply = functools.partial(
    pl.kernel,
    out_type=jax.ShapeDtypeStruct((B, 3, W, W), jnp.float32),
    mesh=_mesh,
    scratch_types=[
        pltpu.VMEM((NLUT,), jnp.float32),
        pltpu.VMEM((NLUT,), jnp.float32),
        pltpu.VMEM((NLUT,), jnp.float32),
        pltpu.VMEM((2, 4, ROWS_PER_CHUNK, W), jnp.float32),
        pltpu.VMEM((2, 3, ROWS_PER_CHUNK, W), jnp.float32),
        pltpu.SemaphoreType.DMA((2,)),
        pltpu.SemaphoreType.DMA((2,)),
    ],
    compiler_params=pltpu.CompilerParams(needs_layout_passes=False),
)(_lut_body)


@jax.jit
def kernel(x, LUT_en):
    lutf = LUT_en.reshape(3, NLUT)
    return _lut_apply(x, lutf[0], lutf[1], lutf[2])


# speedup vs baseline: 1.2188x; 1.1397x over previous
import functools

import jax
import jax.numpy as jnp
import numpy as np
from jax import lax
from jax.experimental import pallas as pl
from jax.experimental.pallas import tpu as pltpu
from jax.experimental.pallas import tpu_sc as plsc

DIM = 17
D2 = DIM * DIM
D3 = DIM * DIM * DIM
NLUT = 2 * D3
B = 16
W = 512
PIX = W * W
CHUNK = 4096
ROWS_PER_CHUNK = CHUNK // W
NWORKERS = 32
CHUNKS_PER_BATCH = PIX // CHUNK
TOTAL_CHUNKS = B * CHUNKS_PER_BATCH
CHUNKS_PER_WORKER = TOTAL_CHUNKS // NWORKERS

_RSCALE = np.float32((DIM - 1) / 1.000001)
_URSCALE = np.float32(1.0 / 1.000001)


def _compute_chunk(lutR, lutG, lutB, xin, yout):
    def row_body(rr, carry2):
        @plsc.parallel_loop(0, W // 16, unroll=1)
        def px(j):
            s = pl.ds(j * 16, 16)
            u = xin[0, rr, s]
            r = xin[1, rr, s]
            g = xin[2, rr, s]
            bl = xin[3, rr, s]
            rf = r * _RSCALE
            gf = g * _RSCALE
            bf = bl * _RSCALE
            ri = jnp.clip(rf.astype(jnp.int32), 0, DIM - 2)
            gi = jnp.clip(gf.astype(jnp.int32), 0, DIM - 2)
            bi = jnp.clip(bf.astype(jnp.int32), 0, DIM - 2)
            dr = rf - ri.astype(jnp.float32)
            dg = gf - gi.astype(jnp.float32)
            db = bf - bi.astype(jnp.float32)
            du = u * _URSCALE
            base = ri * D2 + gi * DIM + bi
            w_k = (1.0 - db, db)
            w_j = (1.0 - dg, dg)
            w_i = (1.0 - dr, dr)
            acc = [[None] * 3, [None] * 3]
            for di in (0, 1):
                for dj in (0, 1):
                    wij = w_i[di] * w_j[dj]
                    for dk in (0, 1):
                        wijk = wij * w_k[dk]
                        idx0 = base + (di * D2 + dj * DIM + dk)
                        idx1 = idx0 + D3
                        for p, idx in ((0, idx0), (1, idx1)):
                            for c, lut in enumerate((lutR, lutG, lutB)):
                                v = wijk * plsc.load_gather(lut, [idx])
                                a = acc[p][c]
                                acc[p][c] = v if a is None else a + v
            omu = 1.0 - du
            yout[0, rr, s] = omu * acc[0][0] + du * acc[1][0]
            yout[1, rr, s] = omu * acc[0][1] + du * acc[1][1]
            yout[2, rr, s] = omu * acc[0][2] + du * acc[1][2]
        return carry2

    lax.fori_loop(0, ROWS_PER_CHUNK, row_body, 0)


def _lut_body(x_h, lr_h, lg_h, lb_h, out_h,
              lutR, lutG, lutB, xin, yout, si, so):
    pltpu.sync_copy(lr_h, lutR)
    pltpu.sync_copy(lg_h, lutG)
    pltpu.sync_copy(lb_h, lutB)

    cid = lax.axis_index("c")
    sid = lax.axis_index("s")
    wid = sid * 2 + cid
    t_base = wid * CHUNKS_PER_WORKER

    def src_in(t):
        b = t // CHUNKS_PER_BATCH
        row0 = (t % CHUNKS_PER_BATCH) * ROWS_PER_CHUNK
        return x_h.at[b, :, pl.ds(row0, ROWS_PER_CHUNK), :]

    def dst_out(t):
        b = t // CHUNKS_PER_BATCH
        row0 = (t % CHUNKS_PER_BATCH) * ROWS_PER_CHUNK
        return out_h.at[b, :, pl.ds(row0, ROWS_PER_CHUNK), :]

    pltpu.async_copy(src_in(t_base), xin.at[0], si.at[0])

    def body(tl, carry):
        t = t_base + tl
        par = tl & 1
        npar = 1 - par
        tnext = t_base + jnp.minimum(tl + 1, CHUNKS_PER_WORKER - 1)
        pltpu.async_copy(src_in(tnext), xin.at[npar], si.at[npar])
        pltpu.make_async_copy(src_in(t), xin.at[par], si.at[par]).wait()

        @pl.when(tl > 1)
        def _():
            pltpu.make_async_copy(yout.at[par], dst_out(t), so.at[par]).wait()

        _compute_chunk(lutR, lutG, lutB, xin.at[par], yout.at[par])
        pltpu.async_copy(yout.at[par], dst_out(t), so.at[par])
        return carry

    lax.fori_loop(0, CHUNKS_PER_WORKER, body, 0)
    pltpu.make_async_copy(src_in(t_base), xin.at[0], si.at[0]).wait()
    pltpu.make_async_copy(yout.at[0], dst_out(t_base), so.at[0]).wait()
    pltpu.make_async_copy(yout.at[1], dst_out(t_base), so.at[1]).wait()


_mesh = plsc.VectorSubcoreMesh(core_axis_name="c", subcore_axis_name="s")

_lut_apply = functools.partial(
    pl.kernel,
    out_type=jax.ShapeDtypeStruct((B, 3, W, W), jnp.float32),
    mesh=_mesh,
    scratch_types=[
        pltpu.VMEM((NLUT,), jnp.float32),
        pltpu.VMEM((NLUT,), jnp.float32),
        pltpu.VMEM((NLUT,), jnp.float32),
        pltpu.VMEM((2, 4, ROWS_PER_CHUNK, W), jnp.float32),
        pltpu.VMEM((2, 3, ROWS_PER_CHUNK, W), jnp.float32),
        pltpu.SemaphoreType.DMA((2,)),
        pltpu.SemaphoreType.DMA((2,)),
    ],
    compiler_params=pltpu.CompilerParams(needs_layout_passes=False),
)(_lut_body)


@jax.jit
def kernel(x, LUT_en):
    lutf = LUT_en.reshape(3, NLUT)
    return _lut_apply(x, lutf[0], lutf[1], lutf[2])


# speedup vs baseline: 1.3901x; 1.1405x over previous
import functools

import jax
import jax.numpy as jnp
import numpy as np
from jax import lax
from jax.experimental import pallas as pl
from jax.experimental.pallas import tpu as pltpu
from jax.experimental.pallas import tpu_sc as plsc

DIM = 17
D2 = DIM * DIM
D3 = DIM * DIM * DIM
NLUT = 2 * D3
B = 16
W = 512
PIX = W * W
CHUNK = 4096
ROWS_PER_CHUNK = CHUNK // W
NWORKERS = 32
CHUNKS_PER_BATCH = PIX // CHUNK
TOTAL_CHUNKS = B * CHUNKS_PER_BATCH
CHUNKS_PER_WORKER = TOTAL_CHUNKS // NWORKERS

_RSCALE = np.float32((DIM - 1) / 1.000001)
_URSCALE = np.float32(1.0 / 1.000001)


def _compute_chunk(lutR, lutG, lutB, xin, yout):
    def row_body(rr, carry2):
        @plsc.parallel_loop(0, W // 16, unroll=1)
        def px(j):
            s = pl.ds(j * 16, 16)
            u = xin[0, rr, s]
            r = xin[1, rr, s]
            g = xin[2, rr, s]
            bl = xin[3, rr, s]
            rf = r * _RSCALE
            gf = g * _RSCALE
            bf = bl * _RSCALE
            ri = rf.astype(jnp.int32)
            gi = gf.astype(jnp.int32)
            bi = bf.astype(jnp.int32)
            dr = rf - ri.astype(jnp.float32)
            dg = gf - gi.astype(jnp.float32)
            db = bf - bi.astype(jnp.float32)
            du = u * _URSCALE
            base = ri * D2 + gi * DIM + bi
            w_k = (1.0 - db, db)
            w_j = (1.0 - dg, dg)
            w_i = (1.0 - dr, dr)
            acc = [[None] * 3, [None] * 3]
            for di in (0, 1):
                for dj in (0, 1):
                    wij = w_i[di] * w_j[dj]
                    for dk in (0, 1):
                        wijk = wij * w_k[dk]
                        idx0 = base + (di * D2 + dj * DIM + dk)
                        idx1 = idx0 + D3
                        for p, idx in ((0, idx0), (1, idx1)):
                            for c, lut in enumerate((lutR, lutG, lutB)):
                                v = wijk * plsc.load_gather(lut, [idx])
                                a = acc[p][c]
                                acc[p][c] = v if a is None else a + v
            omu = 1.0 - du
            yout[0, rr, s] = omu * acc[0][0] + du * acc[1][0]
            yout[1, rr, s] = omu * acc[0][1] + du * acc[1][1]
            yout[2, rr, s] = omu * acc[0][2] + du * acc[1][2]
        return carry2

    lax.fori_loop(0, ROWS_PER_CHUNK, row_body, 0)


def _lut_body(x_h, lr_h, lg_h, lb_h, out_h,
              lutR, lutG, lutB, xin, yout, si, so):
    pltpu.sync_copy(lr_h, lutR)
    pltpu.sync_copy(lg_h, lutG)
    pltpu.sync_copy(lb_h, lutB)

    cid = lax.axis_index("c")
    sid = lax.axis_index("s")
    wid = sid * 2 + cid
    t_base = wid * CHUNKS_PER_WORKER

    def src_in(t):
        b = t // CHUNKS_PER_BATCH
        row0 = (t % CHUNKS_PER_BATCH) * ROWS_PER_CHUNK
        return x_h.at[b, :, pl.ds(row0, ROWS_PER_CHUNK), :]

    def dst_out(t):
        b = t // CHUNKS_PER_BATCH
        row0 = (t % CHUNKS_PER_BATCH) * ROWS_PER_CHUNK
        return out_h.at[b, :, pl.ds(row0, ROWS_PER_CHUNK), :]

    pltpu.async_copy(src_in(t_base), xin.at[0], si.at[0])

    def body(tl, carry):
        t = t_base + tl
        par = tl & 1
        npar = 1 - par
        tnext = t_base + jnp.minimum(tl + 1, CHUNKS_PER_WORKER - 1)
        pltpu.async_copy(src_in(tnext), xin.at[npar], si.at[npar])
        pltpu.make_async_copy(src_in(t), xin.at[par], si.at[par]).wait()

        @pl.when(tl > 1)
        def _():
            pltpu.make_async_copy(yout.at[par], dst_out(t), so.at[par]).wait()

        _compute_chunk(lutR, lutG, lutB, xin.at[par], yout.at[par])
        pltpu.async_copy(yout.at[par], dst_out(t), so.at[par])
        return carry

    lax.fori_loop(0, CHUNKS_PER_WORKER, body, 0)
    pltpu.make_async_copy(src_in(t_base), xin.at[0], si.at[0]).wait()
    pltpu.make_async_copy(yout.at[0], dst_out(t_base), so.at[0]).wait()
    pltpu.make_async_copy(yout.at[1], dst_out(t_base), so.at[1]).wait()


_mesh = plsc.VectorSubcoreMesh(core_axis_name="c", subcore_axis_name="s")

_lut_apply = functools.partial(
    pl.kernel,
    out_type=jax.ShapeDtypeStruct((B, 3, W, W), jnp.float32),
    mesh=_mesh,
    scratch_types=[
        pltpu.VMEM((NLUT,), jnp.float32),
        pltpu.VMEM((NLUT,), jnp.float32),
        pltpu.VMEM((NLUT,), jnp.float32),
        pltpu.VMEM((2, 4, ROWS_PER_CHUNK, W), jnp.float32),
        pltpu.VMEM((2, 3, ROWS_PER_CHUNK, W), jnp.float32),
        pltpu.SemaphoreType.DMA((2,)),
        pltpu.SemaphoreType.DMA((2,)),
    ],
    compiler_params=pltpu.CompilerParams(needs_layout_passes=False),
)(_lut_body)


@jax.jit
def kernel(x, LUT_en):
    lutf = LUT_en.reshape(3, NLUT)
    return _lut_apply(x, lutf[0], lutf[1], lutf[2])


# speedup vs baseline: 1.4020x; 1.0085x over previous
import functools

import jax
import jax.numpy as jnp
import numpy as np
from jax import lax
from jax.experimental import pallas as pl
from jax.experimental.pallas import tpu as pltpu
from jax.experimental.pallas import tpu_sc as plsc

DIM = 17
D2 = DIM * DIM
D3 = DIM * DIM * DIM
NLUT = 2 * D3
B = 16
W = 512
PIX = W * W
CHUNK = 4096
ROWS_PER_CHUNK = CHUNK // W
NWORKERS = 32
CHUNKS_PER_BATCH = PIX // CHUNK
TOTAL_CHUNKS = B * CHUNKS_PER_BATCH
CHUNKS_PER_WORKER = TOTAL_CHUNKS // NWORKERS

_RSCALE = np.float32((DIM - 1) / 1.000001)
_URSCALE = np.float32(1.0 / 1.000001)


def _compute_chunk(lutR, lutG, lutB, xin, yout):
    def row_body(rr, carry2):
        @plsc.parallel_loop(0, W // 16, unroll=1)
        def px(j):
            s = pl.ds(j * 16, 16)
            u = xin[0, rr, s]
            r = xin[1, rr, s]
            g = xin[2, rr, s]
            bl = xin[3, rr, s]
            rf = r * _RSCALE
            gf = g * _RSCALE
            bf = bl * _RSCALE
            ri = rf.astype(jnp.int32)
            gi = gf.astype(jnp.int32)
            bi = bf.astype(jnp.int32)
            dr = rf - ri.astype(jnp.float32)
            dg = gf - gi.astype(jnp.float32)
            db = bf - bi.astype(jnp.float32)
            du = u * _URSCALE
            base = ri * D2 + gi * DIM + bi
            w_k = (1.0 - db, db)
            w_j = (1.0 - dg, dg)
            w_i = (1.0 - dr, dr)
            acc = [[None] * 3, [None] * 3]
            for di in (0, 1):
                for dj in (0, 1):
                    wij = w_i[di] * w_j[dj]
                    for dk in (0, 1):
                        wijk = wij * w_k[dk]
                        idx0 = base + (di * D2 + dj * DIM + dk)
                        idx1 = idx0 + D3
                        for p, idx in ((0, idx0), (1, idx1)):
                            for c, lut in enumerate((lutR, lutG, lutB)):
                                v = wijk * plsc.load_gather(lut, [idx])
                                a = acc[p][c]
                                acc[p][c] = v if a is None else a + v
            omu = 1.0 - du
            yout[0, rr, s] = omu * acc[0][0] + du * acc[1][0]
            yout[1, rr, s] = omu * acc[0][1] + du * acc[1][1]
            yout[2, rr, s] = omu * acc[0][2] + du * acc[1][2]
        return carry2

    lax.fori_loop(0, ROWS_PER_CHUNK, row_body, 0)


def _lut_body(x_h, lr_h, lg_h, lb_h, out_h,
              lutR, lutG, lutB, xin, yout, si, so):
    cid = lax.axis_index("c")
    sid = lax.axis_index("s")
    wid = sid * 2 + cid
    t_base = wid * CHUNKS_PER_WORKER

    def src_in(t):
        b = t // CHUNKS_PER_BATCH
        row0 = (t % CHUNKS_PER_BATCH) * ROWS_PER_CHUNK
        return x_h.at[b, :, pl.ds(row0, ROWS_PER_CHUNK), :]

    def dst_out(t):
        b = t // CHUNKS_PER_BATCH
        row0 = (t % CHUNKS_PER_BATCH) * ROWS_PER_CHUNK
        return out_h.at[b, :, pl.ds(row0, ROWS_PER_CHUNK), :]

    pltpu.async_copy(src_in(t_base), xin.at[0], si.at[0])

    pltpu.async_copy(lr_h, lutR, so.at[0])
    pltpu.async_copy(lg_h, lutG, so.at[0])
    pltpu.async_copy(lb_h, lutB, so.at[0])
    pltpu.make_async_copy(lr_h, lutR, so.at[0]).wait()
    pltpu.make_async_copy(lg_h, lutG, so.at[0]).wait()
    pltpu.make_async_copy(lb_h, lutB, so.at[0]).wait()

    def body(tl, carry):
        t = t_base + tl
        par = tl & 1
        npar = 1 - par
        tnext = t_base + jnp.minimum(tl + 1, CHUNKS_PER_WORKER - 1)
        pltpu.async_copy(src_in(tnext), xin.at[npar], si.at[npar])
        pltpu.make_async_copy(src_in(t), xin.at[par], si.at[par]).wait()

        @pl.when(tl > 1)
        def _():
            pltpu.make_async_copy(yout.at[par], dst_out(t), so.at[par]).wait()

        _compute_chunk(lutR, lutG, lutB, xin.at[par], yout.at[par])
        pltpu.async_copy(yout.at[par], dst_out(t), so.at[par])
        return carry

    lax.fori_loop(0, CHUNKS_PER_WORKER, body, 0)
    pltpu.make_async_copy(src_in(t_base), xin.at[0], si.at[0]).wait()
    pltpu.make_async_copy(yout.at[0], dst_out(t_base), so.at[0]).wait()
    pltpu.make_async_copy(yout.at[1], dst_out(t_base), so.at[1]).wait()


_mesh = plsc.VectorSubcoreMesh(core_axis_name="c", subcore_axis_name="s")

_lut_apply = functools.partial(
    pl.kernel,
    out_type=jax.ShapeDtypeStruct((B, 3, W, W), jnp.float32),
    mesh=_mesh,
    scratch_types=[
        pltpu.VMEM((NLUT,), jnp.float32),
        pltpu.VMEM((NLUT,), jnp.float32),
        pltpu.VMEM((NLUT,), jnp.float32),
        pltpu.VMEM((2, 4, ROWS_PER_CHUNK, W), jnp.float32),
        pltpu.VMEM((2, 3, ROWS_PER_CHUNK, W), jnp.float32),
        pltpu.SemaphoreType.DMA((2,)),
        pltpu.SemaphoreType.DMA((2,)),
    ],
    compiler_params=pltpu.CompilerParams(needs_layout_passes=False),
)(_lut_body)


@jax.jit
def kernel(x, LUT_en):
    lutf = LUT_en.reshape(3, NLUT)
    return _lut_apply(x, lutf[0], lutf[1], lutf[2])
